# baseline (speedup 1.0000x reference)
import functools

import jax
import jax.numpy as jnp
from jax.experimental import pallas as pl
from jax.experimental.pallas import tpu as pltpu

_B, _S, _D = 1, 2048, 1024
_E, _TOPK, _F = 8, 2, 512
_N = _B * _S
_A = _N * _TOPK

_BLK = 128
_NBLK = _A // _BLK
_G = _NBLK + _E - 1

_NEG = float("-inf")


def _router_body(x_ref, rw_ref, logits_ref, i12_ref, p12_ref):
    x = x_ref[...]
    rw = rw_ref[...]
    logits = jax.lax.dot_general(
        x, rw, (((1,), (1,)), ((), ())), preferred_element_type=jnp.float32)
    logits_ref[...] = logits
    logits_t = jax.lax.dot_general(
        rw, x, (((1,), (1,)), ((), ())), preferred_element_type=jnp.float32)
    idx = jax.lax.broadcasted_iota(jnp.int32, (_E, _N), 0)
    m1 = jnp.max(logits_t, axis=0, keepdims=True)
    i1 = jnp.min(jnp.where(logits_t == m1, idx, _E), axis=0, keepdims=True)
    l2 = jnp.where(idx == i1, _NEG, logits_t)
    m2 = jnp.max(l2, axis=0, keepdims=True)
    i2 = jnp.min(jnp.where(l2 == m2, idx, _E), axis=0, keepdims=True)
    t = jnp.exp(m2 - m1)
    p1 = 1.0 / (1.0 + t)
    i12_ref[...] = jnp.concatenate([i1, i2], axis=0)
    p12_ref[...] = jnp.concatenate([p1, t * p1], axis=0)


def _ffn_body(blk_ref, ex_ref, off_ref, x_ref, gw_ref, uw_ref, dw_ref, y_ref):
    t = pl.program_id(0)
    prev_blk = blk_ref[jnp.maximum(t - 1, 0)]
    first = jnp.logical_or(t == 0, blk_ref[t] != prev_blk)
    e = ex_ref[t]

    @pl.when(first)
    def _():
        y_ref[...] = jnp.zeros_like(y_ref)

    @pl.when(e < _E)
    def _():
        x = x_ref[...]
        g = jax.lax.dot_general(
            x, gw_ref[0], (((1,), (1,)), ((), ())),
            preferred_element_type=jnp.float32)
        u = jax.lax.dot_general(
            x, uw_ref[0], (((1,), (1,)), ((), ())),
            preferred_element_type=jnp.float32)
        h = (g * jax.lax.logistic(g)) * u
        d = jax.lax.dot_general(
            h, dw_ref[0], (((1,), (1,)), ((), ())),
            preferred_element_type=jnp.float32)
        start = off_ref[e]
        end = off_ref[e + 1]
        rowg = blk_ref[t] * _BLK + jax.lax.broadcasted_iota(
            jnp.int32, (_BLK, 1), 0)
        mask = jnp.logical_and(rowg >= start, rowg < end)
        y_ref[...] += jnp.where(mask, d, 0.0)


def _combine_body(y_ref, p_ref, out_ref):
    p = p_ref[...]
    out_ref[...] = p[:, 0:1] * y_ref[0] + p[:, 1:2] * y_ref[1]


def _pair_schedule(off):
    jj = jnp.arange(_NBLK, dtype=jnp.int32)[:, None]
    ee = jnp.arange(_E, dtype=jnp.int32)[None, :]
    lo = off[:-1][None, :]
    hi = off[1:][None, :]
    active = jnp.logical_and(lo < (jj + 1) * _BLK, hi > jj * _BLK)
    sidx = jnp.nonzero(active.reshape(-1), size=_G, fill_value=-1)[0]
    valid = sidx >= 0
    nvalid = jnp.sum(valid.astype(jnp.int32))
    last = sidx[jnp.maximum(nvalid - 1, 0)]
    blk = jnp.where(valid, sidx // _E, last // _E).astype(jnp.int32)
    ex = jnp.where(valid, sidx % _E, _E).astype(jnp.int32)
    return blk, ex


@jax.jit
def kernel(hidden_states, router_w, gate_w, up_w, down_w):
    x = hidden_states.reshape(_N, _D)
    logits, i12, p12 = pl.pallas_call(
        _router_body,
        out_shape=(
            jax.ShapeDtypeStruct((_N, _E), jnp.float32),
            jax.ShapeDtypeStruct((_TOPK, _N), jnp.int32),
            jax.ShapeDtypeStruct((_TOPK, _N), jnp.float32),
        ),
    )(x, router_w)

    eid = i12.reshape(_A)
    order = jnp.argsort(eid, stable=True)
    pos = jnp.argsort(order)
    x_sorted = x[order % _N]
    counts = jnp.bincount(eid, length=_E)
    off = jnp.concatenate(
        [jnp.zeros((1,), jnp.int32), jnp.cumsum(counts).astype(jnp.int32)])
    off16 = jnp.concatenate([off, jnp.full((7,), _A, jnp.int32)])
    blk, ex = _pair_schedule(off)

    y_sorted = pl.pallas_call(
        _ffn_body,
        grid_spec=pltpu.PrefetchScalarGridSpec(
            num_scalar_prefetch=3,
            grid=(_G,),
            in_specs=[
                pl.BlockSpec((_BLK, _D), lambda t, b, e, o: (b[t], 0)),
                pl.BlockSpec((1, _F, _D),
                             lambda t, b, e, o: (jnp.minimum(e[t], _E - 1), 0, 0)),
                pl.BlockSpec((1, _F, _D),
                             lambda t, b, e, o: (jnp.minimum(e[t], _E - 1), 0, 0)),
                pl.BlockSpec((1, _D, _F),
                             lambda t, b, e, o: (jnp.minimum(e[t], _E - 1), 0, 0)),
            ],
            out_specs=pl.BlockSpec((_BLK, _D), lambda t, b, e, o: (b[t], 0)),
        ),
        out_shape=jax.ShapeDtypeStruct((_A, _D), jnp.float32),
    )(blk, ex, off16, x_sorted, gate_w, up_w, down_w)

    y_asgn = y_sorted[pos].reshape(_TOPK, _N, _D)

    out = pl.pallas_call(
        _combine_body,
        out_shape=jax.ShapeDtypeStruct((_N, _D), jnp.float32),
    )(y_asgn, p12.T)

    return out.reshape(_B, _S, _D), logits


# speedup vs baseline: 1.1253x; 1.1253x over previous
import functools

import jax
import jax.numpy as jnp
from jax import lax
from jax.experimental import pallas as pl
from jax.experimental.pallas import tpu as pltpu
from jax.experimental.pallas import tpu_sc as plsc

_B, _S, _D = 1, 2048, 1024
_E, _TOPK, _F = 8, 2, 512
_N = _B * _S
_A = _N * _TOPK

_BLK = 128
_NBLK = _A // _BLK
_G = _NBLK + _E - 1

_NW = 32
_CHUNK = _A // _NW
_HALF = _CHUNK // 2

_NEG = float("-inf")


def _router_body(x_ref, rw_ref, logits_ref, i12_ref, p12_ref):
    x = x_ref[...]
    rw = rw_ref[...]
    logits = jax.lax.dot_general(
        x, rw, (((1,), (1,)), ((), ())), preferred_element_type=jnp.float32)
    logits_ref[...] = logits
    logits_t = jax.lax.dot_general(
        rw, x, (((1,), (1,)), ((), ())), preferred_element_type=jnp.float32)
    idx = jax.lax.broadcasted_iota(jnp.int32, (_E, _N), 0)
    m1 = jnp.max(logits_t, axis=0, keepdims=True)
    i1 = jnp.min(jnp.where(logits_t == m1, idx, _E), axis=0, keepdims=True)
    l2 = jnp.where(idx == i1, _NEG, logits_t)
    m2 = jnp.max(l2, axis=0, keepdims=True)
    i2 = jnp.min(jnp.where(l2 == m2, idx, _E), axis=0, keepdims=True)
    t = jnp.exp(m2 - m1)
    p1 = 1.0 / (1.0 + t)
    i12_ref[...] = jnp.concatenate([i1, i2], axis=0)
    p12_ref[...] = jnp.concatenate([p1, t * p1], axis=0)


def _histo_chunk(eid_all, acc, iota, c):
    for v in range(_CHUNK // 16):
        ev = eid_all[pl.ds(c * _CHUNK + v * 16, 16)]
        for e in range(_E):
            pc = jnp.sum(jnp.where(ev == e, 1, 0))
            acc = acc + jnp.where(iota == e, pc, 0)
    return acc


def _dispatch_kernel(eid_hbm, x_hbm, pos_out, off_out, xs_out,
                     eid_v, pos_v, off_v, xbuf, sem):
    wid = lax.axis_index("s") * 2 + lax.axis_index("c")
    iota = lax.iota(jnp.int32, 16)

    pltpu.sync_copy(eid_hbm, eid_v)
    eid_all = eid_v

    zero16 = jnp.zeros((16,), jnp.int32)
    h = lax.fori_loop(
        0, _NW, lambda c, acc: _histo_chunk(eid_all, acc, iota, c), zero16)
    offv = plsc.cumsum(h) - h

    @pl.when(wid == 0)
    def _():
        off_v[...] = offv
        pltpu.sync_copy(off_v, off_out)

    pre = lax.fori_loop(
        0, wid, lambda c, acc: _histo_chunk(eid_all, acc, iota, c), zero16)

    hv = offv + pre
    for v in range(_CHUNK // 16):
        ev = eid_all[pl.ds(wid * _CHUNK + v * 16, 16)]
        pos_lanes = jnp.zeros((16,), jnp.int32)
        hv_new = hv
        for e in range(_E):
            mask = ev == e
            m1 = jnp.where(mask, 1, 0)
            incl = plsc.cumsum(m1)
            prefix_e = jnp.sum(jnp.where(iota == e, hv, 0))
            cnt_e = jnp.sum(m1)
            pos_lanes = jnp.where(mask, prefix_e + incl - 1, pos_lanes)
            hv_new = hv_new + jnp.where(iota == e, cnt_e, 0)
        hv = hv_new
        pos_v[v // 4, pl.ds((v % 4) * 16, 16)] = pos_lanes
    pltpu.sync_copy(pos_v, pos_out.at[wid])

    for k in range(2):
        tok_base = (wid % (_NW // 2)) * _CHUNK + k * _HALF
        pltpu.sync_copy(x_hbm.at[pl.ds(tok_base, _HALF)], xbuf)
        pltpu.async_copy(xbuf, xs_out.at[pos_v.at[k]], sem).wait()


def _ffn_body(blk_ref, ex_ref, off_ref, x_ref, gw_ref, uw_ref, dw_ref, y_ref):
    t = pl.program_id(0)
    prev_blk = blk_ref[jnp.maximum(t - 1, 0)]
    first = jnp.logical_or(t == 0, blk_ref[t] != prev_blk)
    e = ex_ref[t]

    @pl.when(first)
    def _():
        y_ref[...] = jnp.zeros_like(y_ref)

    @pl.when(e < _E)
    def _():
        x = x_ref[...]
        g = jax.lax.dot_general(
            x, gw_ref[0], (((1,), (1,)), ((), ())),
            preferred_element_type=jnp.float32)
        u = jax.lax.dot_general(
            x, uw_ref[0], (((1,), (1,)), ((), ())),
            preferred_element_type=jnp.float32)
        h = (g * jax.lax.logistic(g)) * u
        d = jax.lax.dot_general(
            h, dw_ref[0], (((1,), (1,)), ((), ())),
            preferred_element_type=jnp.float32)
        start = off_ref[e]
        end = off_ref[e + 1]
        rowg = blk_ref[t] * _BLK + jax.lax.broadcasted_iota(
            jnp.int32, (_BLK, 1), 0)
        mask = jnp.logical_and(rowg >= start, rowg < end)
        y_ref[...] += jnp.where(mask, d, 0.0)


def _unsort_kernel(ys_hbm, pos_hbm, ya_out, pos_v, ybuf, sem):
    wid = lax.axis_index("s") * 2 + lax.axis_index("c")
    pltpu.sync_copy(pos_hbm.at[wid], pos_v)
    for k in range(2):
        pltpu.async_copy(ys_hbm.at[pos_v.at[k]], ybuf, sem).wait()
        pltpu.sync_copy(
            ybuf, ya_out.at[pl.ds(wid * _CHUNK + k * _HALF, _HALF)])


def _combine_body(y_ref, p_ref, out_ref):
    p = p_ref[...]
    out_ref[...] = p[:, 0:1] * y_ref[0] + p[:, 1:2] * y_ref[1]


def _pair_schedule(off):
    jj = jnp.arange(_NBLK, dtype=jnp.int32)[:, None]
    ee = jnp.arange(_E, dtype=jnp.int32)[None, :]
    del ee
    lo = off[:-1][None, :]
    hi = off[1:][None, :]
    active = jnp.logical_and(lo < (jj + 1) * _BLK, hi > jj * _BLK)
    sidx = jnp.nonzero(active.reshape(-1), size=_G, fill_value=-1)[0]
    valid = sidx >= 0
    nvalid = jnp.sum(valid.astype(jnp.int32))
    last = sidx[jnp.maximum(nvalid - 1, 0)]
    blk = jnp.where(valid, sidx // _E, last // _E).astype(jnp.int32)
    ex = jnp.where(valid, sidx % _E, _E).astype(jnp.int32)
    return blk, ex


_sc_mesh = plsc.VectorSubcoreMesh(core_axis_name="c", subcore_axis_name="s")

_dispatch = functools.partial(
    pl.kernel, _dispatch_kernel, mesh=_sc_mesh,
    compiler_params=pltpu.CompilerParams(needs_layout_passes=False),
    out_type=(
        jax.ShapeDtypeStruct((_NW, 2, _HALF), jnp.int32),
        jax.ShapeDtypeStruct((16,), jnp.int32),
        jax.ShapeDtypeStruct((_A, _D), jnp.float32),
    ),
    scratch_types=[
        pltpu.VMEM((_A,), jnp.int32),
        pltpu.VMEM((2, _HALF), jnp.int32),
        pltpu.VMEM((16,), jnp.int32),
        pltpu.VMEM((_HALF, _D), jnp.float32),
        pltpu.SemaphoreType.DMA,
    ],
)

_unsort = functools.partial(
    pl.kernel, _unsort_kernel, mesh=_sc_mesh,
    out_type=jax.ShapeDtypeStruct((_A, _D), jnp.float32),
    scratch_types=[
        pltpu.VMEM((2, _HALF), jnp.int32),
        pltpu.VMEM((_HALF, _D), jnp.float32),
        pltpu.SemaphoreType.DMA,
    ],
)


@jax.jit
def kernel(hidden_states, router_w, gate_w, up_w, down_w):
    x = hidden_states.reshape(_N, _D)
    logits, i12, p12 = pl.pallas_call(
        _router_body,
        out_shape=(
            jax.ShapeDtypeStruct((_N, _E), jnp.float32),
            jax.ShapeDtypeStruct((_TOPK, _N), jnp.int32),
            jax.ShapeDtypeStruct((_TOPK, _N), jnp.float32),
        ),
    )(x, router_w)

    eid = i12.reshape(_A)
    pos3, off16, x_sorted = _dispatch()(eid, x)
    off = off16[:_E + 1]
    blk, ex = _pair_schedule(off)

    y_sorted = pl.pallas_call(
        _ffn_body,
        grid_spec=pltpu.PrefetchScalarGridSpec(
            num_scalar_prefetch=3,
            grid=(_G,),
            in_specs=[
                pl.BlockSpec((_BLK, _D), lambda t, b, e, o: (b[t], 0)),
                pl.BlockSpec((1, _F, _D),
                             lambda t, b, e, o: (jnp.minimum(e[t], _E - 1), 0, 0)),
                pl.BlockSpec((1, _F, _D),
                             lambda t, b, e, o: (jnp.minimum(e[t], _E - 1), 0, 0)),
                pl.BlockSpec((1, _D, _F),
                             lambda t, b, e, o: (jnp.minimum(e[t], _E - 1), 0, 0)),
            ],
            out_specs=pl.BlockSpec((_BLK, _D), lambda t, b, e, o: (b[t], 0)),
        ),
        out_shape=jax.ShapeDtypeStruct((_A, _D), jnp.float32),
    )(blk, ex, off16, x_sorted, gate_w, up_w, down_w)

    y_asgn = _unsort()(y_sorted, pos3).reshape(_TOPK, _N, _D)

    out = pl.pallas_call(
        _combine_body,
        out_shape=jax.ShapeDtypeStruct((_N, _D), jnp.float32),
    )(y_asgn, p12.T)

    return out.reshape(_B, _S, _D), logits


# speedup vs baseline: 1.4804x; 1.3156x over previous
import functools

import jax
import jax.numpy as jnp
from jax import lax
from jax.experimental import pallas as pl
from jax.experimental.pallas import tpu as pltpu
from jax.experimental.pallas import tpu_sc as plsc

_B, _S, _D = 1, 2048, 1024
_E, _TOPK, _F = 8, 2, 512
_N = _B * _S
_A = _N * _TOPK

_BLK = 256
_AP = _A + _E * _BLK
_NBLKP = _AP // _BLK

_NW = 32
_CHUNK = _A // _NW
_HALF = _CHUNK // 2

_NEG = float("-inf")


def _router_body(x_ref, rw_ref, logits_ref, i12_ref, p12_ref):
    x = x_ref[...]
    rw = rw_ref[...]
    logits = jax.lax.dot_general(
        x, rw, (((1,), (1,)), ((), ())), preferred_element_type=jnp.float32)
    logits_ref[...] = logits
    logits_t = jax.lax.dot_general(
        rw, x, (((1,), (1,)), ((), ())), preferred_element_type=jnp.float32)
    idx = jax.lax.broadcasted_iota(jnp.int32, (_E, _N), 0)
    m1 = jnp.max(logits_t, axis=0, keepdims=True)
    i1 = jnp.min(jnp.where(logits_t == m1, idx, _E), axis=0, keepdims=True)
    l2 = jnp.where(idx == i1, _NEG, logits_t)
    m2 = jnp.max(l2, axis=0, keepdims=True)
    i2 = jnp.min(jnp.where(l2 == m2, idx, _E), axis=0, keepdims=True)
    t = jnp.exp(m2 - m1)
    p1 = 1.0 / (1.0 + t)
    i12_ref[...] = jnp.concatenate([i1, i2], axis=0)
    p12_ref[...] = jnp.concatenate([p1, t * p1], axis=0)


def _histo_chunk(eid_all, hist_ref, ones16, c):
    for v in range(_CHUNK // 16):
        ev = eid_all[pl.ds(c * _CHUNK + v * 16, 16)]
        plsc.addupdate_scatter(hist_ref, [ev], ones16)


def _dispatch_kernel(eid_hbm, x_hbm, pos_out, off_out, xs_out,
                     eid_v, nf_v, pos_v, off_v, xbuf, sem):
    wid = lax.axis_index("s") * 2 + lax.axis_index("c")
    ones16 = jnp.ones((16,), jnp.int32)

    pltpu.sync_copy(eid_hbm, eid_v)
    eid_all = eid_v

    nf_v[...] = jnp.zeros((16,), jnp.int32)

    def hbody(c, carry):
        _histo_chunk(eid_all, nf_v, ones16, c)
        return carry

    lax.fori_loop(0, wid, hbody, 0)
    pre = nf_v[...]
    lax.fori_loop(wid, _NW, hbody, 0)
    h = nf_v[...]
    hp = jnp.bitwise_and(h + (_BLK - 1), -_BLK)
    cum_hp = plsc.cumsum(hp)
    offv = cum_hp - hp

    @pl.when(wid == 0)
    def _():
        off_v[...] = cum_hp
        pltpu.sync_copy(off_v, off_out)

    nf_v[...] = offv + pre
    for v in range(_CHUNK // 16):
        ev = eid_all[pl.ds(wid * _CHUNK + v * 16, 16)]
        base = plsc.load_gather(nf_v, [ev])
        rank = jnp.zeros((16,), jnp.int32)
        for e in range(_E):
            mask = ev == e
            incl = plsc.cumsum(jnp.where(mask, 1, 0))
            rank = jnp.where(mask, incl - 1, rank)
        pos_v[v // 4, pl.ds((v % 4) * 16, 16)] = base + rank
        plsc.addupdate_scatter(nf_v, [ev], ones16)
    pltpu.sync_copy(pos_v, pos_out.at[wid])

    for k in range(2):
        tok_base = (wid % (_NW // 2)) * _CHUNK + k * _HALF
        pltpu.sync_copy(x_hbm.at[pl.ds(tok_base, _HALF)], xbuf)
        pltpu.async_copy(xbuf, xs_out.at[pos_v.at[k]], sem).wait()


def _ffn_body(ex_ref, x_ref, gw_ref, uw_ref, dw_ref, y_ref):
    t = pl.program_id(0)
    e = ex_ref[t]

    @pl.when(e < _E)
    def _():
        x = x_ref[...]
        g = jax.lax.dot_general(
            x, gw_ref[0], (((1,), (1,)), ((), ())),
            preferred_element_type=jnp.float32)
        u = jax.lax.dot_general(
            x, uw_ref[0], (((1,), (1,)), ((), ())),
            preferred_element_type=jnp.float32)
        h = (g * jax.lax.logistic(g)) * u
        y_ref[...] = jax.lax.dot_general(
            h, dw_ref[0], (((1,), (1,)), ((), ())),
            preferred_element_type=jnp.float32)


_SUB = 16
_NSUB = (_N // _NW) // _SUB


def _combine_kernel(ys_hbm, pos_hbm, p12_hbm, out_hbm,
                    pos_v, p_v, ya, yb, ob, sem):
    wid = lax.axis_index("s") * 2 + lax.axis_index("c")
    iota = lax.iota(jnp.int32, 16)
    tok0 = wid * (_N // _NW)
    pltpu.sync_copy(pos_hbm.at[wid // 2, wid % 2], pos_v.at[0])
    pltpu.sync_copy(pos_hbm.at[_NW // 2 + wid // 2, wid % 2], pos_v.at[1])
    pltpu.sync_copy(p12_hbm.at[0, pl.ds(tok0, _N // _NW)], p_v.at[0])
    pltpu.sync_copy(p12_hbm.at[1, pl.ds(tok0, _N // _NW)], p_v.at[1])
    for s in range(_NSUB):
        cpa = pltpu.async_copy(
            ys_hbm.at[pos_v.at[0, pl.ds(s * _SUB, _SUB)]], ya, sem)
        cpb = pltpu.async_copy(
            ys_hbm.at[pos_v.at[1, pl.ds(s * _SUB, _SUB)]], yb, sem)
        cpa.wait()
        cpb.wait()
        pav = p_v[0, pl.ds(s * _SUB, 16)]
        pbv = p_v[1, pl.ds(s * _SUB, 16)]
        pa_l = [jnp.sum(jnp.where(iota == j, pav, 0.0)) for j in range(_SUB)]
        pb_l = [jnp.sum(jnp.where(iota == j, pbv, 0.0)) for j in range(_SUB)]

        def cbody(c, carry):
            sl = pl.ds(c * 16, 16)
            for j in range(_SUB):
                ob[j, sl] = pa_l[j] * ya[j, sl] + pb_l[j] * yb[j, sl]
            return carry

        lax.fori_loop(0, _D // 16, cbody, 0)
        pltpu.sync_copy(ob, out_hbm.at[pl.ds(tok0 + s * _SUB, _SUB)])


def _block_experts(cum_hp):
    jj = jnp.arange(_NBLKP, dtype=jnp.int32)[:, None] * _BLK
    ex = jnp.sum((cum_hp[None, :_E] <= jj).astype(jnp.int32), axis=1)
    return ex.astype(jnp.int32)


_sc_mesh = plsc.VectorSubcoreMesh(core_axis_name="c", subcore_axis_name="s")

_dispatch = functools.partial(
    pl.kernel, _dispatch_kernel, mesh=_sc_mesh,
    compiler_params=pltpu.CompilerParams(needs_layout_passes=False),
    out_type=(
        jax.ShapeDtypeStruct((_NW, 2, _HALF), jnp.int32),
        jax.ShapeDtypeStruct((16,), jnp.int32),
        jax.ShapeDtypeStruct((_AP, _D), jnp.float32),
    ),
    scratch_types=[
        pltpu.VMEM((_A,), jnp.int32),
        pltpu.VMEM((16,), jnp.int32),
        pltpu.VMEM((2, _HALF), jnp.int32),
        pltpu.VMEM((16,), jnp.int32),
        pltpu.VMEM((_HALF, _D), jnp.float32),
        pltpu.SemaphoreType.DMA,
    ],
)

_combine = functools.partial(
    pl.kernel, _combine_kernel, mesh=_sc_mesh,
    compiler_params=pltpu.CompilerParams(needs_layout_passes=False),
    out_type=jax.ShapeDtypeStruct((_N, _D), jnp.float32),
    scratch_types=[
        pltpu.VMEM((2, _N // _NW), jnp.int32),
        pltpu.VMEM((2, _N // _NW), jnp.float32),
        pltpu.VMEM((_SUB, _D), jnp.float32),
        pltpu.VMEM((_SUB, _D), jnp.float32),
        pltpu.VMEM((_SUB, _D), jnp.float32),
        pltpu.SemaphoreType.DMA,
    ],
)


@jax.jit
def kernel(hidden_states, router_w, gate_w, up_w, down_w):
    x = hidden_states.reshape(_N, _D)
    logits, i12, p12 = pl.pallas_call(
        _router_body,
        out_shape=(
            jax.ShapeDtypeStruct((_N, _E), jnp.float32),
            jax.ShapeDtypeStruct((_TOPK, _N), jnp.int32),
            jax.ShapeDtypeStruct((_TOPK, _N), jnp.float32),
        ),
    )(x, router_w)

    eid = i12.reshape(_A)
    pos3, cum_hp, x_sorted = _dispatch()(eid, x)
    ex = _block_experts(cum_hp)

    y_sorted = pl.pallas_call(
        _ffn_body,
        grid_spec=pltpu.PrefetchScalarGridSpec(
            num_scalar_prefetch=1,
            grid=(_NBLKP,),
            in_specs=[
                pl.BlockSpec((_BLK, _D), lambda t, e: (t, 0)),
                pl.BlockSpec((1, _F, _D),
                             lambda t, e: (jnp.minimum(e[t], _E - 1), 0, 0)),
                pl.BlockSpec((1, _F, _D),
                             lambda t, e: (jnp.minimum(e[t], _E - 1), 0, 0)),
                pl.BlockSpec((1, _D, _F),
                             lambda t, e: (jnp.minimum(e[t], _E - 1), 0, 0)),
            ],
            out_specs=pl.BlockSpec((_BLK, _D), lambda t, e: (t, 0)),
        ),
        out_shape=jax.ShapeDtypeStruct((_AP, _D), jnp.float32),
    )(ex, x_sorted, gate_w, up_w, down_w)

    out = _combine()(y_sorted, pos3, p12)

    return out.reshape(_B, _S, _D), logits


# speedup vs baseline: 1.5773x; 1.0655x over previous
import functools

import jax
import jax.numpy as jnp
from jax import lax
from jax.experimental import pallas as pl
from jax.experimental.pallas import tpu as pltpu
from jax.experimental.pallas import tpu_sc as plsc

_B, _S, _D = 1, 2048, 1024
_E, _TOPK, _F = 8, 2, 512
_N = _B * _S
_A = _N * _TOPK

_BLK = 512
_AP = _A + _E * _BLK
_NBLKP = _AP // _BLK

_NW = 32
_CHUNK = _A // _NW
_HALF = _CHUNK // 2

_NEG = float("-inf")


def _router_body(x_ref, rw_ref, logits_ref, i12_ref, p12_ref):
    x = x_ref[...]
    rw = rw_ref[...]
    logits = jax.lax.dot_general(
        x, rw, (((1,), (1,)), ((), ())), preferred_element_type=jnp.float32)
    logits_ref[...] = logits
    logits_t = jax.lax.dot_general(
        rw, x, (((1,), (1,)), ((), ())), preferred_element_type=jnp.float32)
    idx = jax.lax.broadcasted_iota(jnp.int32, (_E, _N), 0)
    m1 = jnp.max(logits_t, axis=0, keepdims=True)
    i1 = jnp.min(jnp.where(logits_t == m1, idx, _E), axis=0, keepdims=True)
    l2 = jnp.where(idx == i1, _NEG, logits_t)
    m2 = jnp.max(l2, axis=0, keepdims=True)
    i2 = jnp.min(jnp.where(l2 == m2, idx, _E), axis=0, keepdims=True)
    t = jnp.exp(m2 - m1)
    p1 = 1.0 / (1.0 + t)
    i12_ref[...] = jnp.concatenate([i1, i2], axis=0)
    p12_ref[...] = jnp.concatenate([p1, t * p1], axis=0)


def _histo_chunk(eid_all, hist_ref, ones16, c):
    for v in range(_CHUNK // 16):
        ev = eid_all[pl.ds(c * _CHUNK + v * 16, 16)]
        plsc.addupdate_scatter(hist_ref, [ev], ones16)


def _dispatch_kernel(eid_hbm, x_hbm, pos_out, off_out, xs_out,
                     eid_v, nf_v, pos_v, off_v, xbuf, sem):
    wid = lax.axis_index("s") * 2 + lax.axis_index("c")
    ones16 = jnp.ones((16,), jnp.int32)

    pltpu.sync_copy(eid_hbm, eid_v)
    eid_all = eid_v

    nf_v[...] = jnp.zeros((16,), jnp.int32)

    def hbody(c, carry):
        _histo_chunk(eid_all, nf_v, ones16, c)
        return carry

    lax.fori_loop(0, wid, hbody, 0)
    pre = nf_v[...]
    lax.fori_loop(wid, _NW, hbody, 0)
    h = nf_v[...]
    hp = jnp.bitwise_and(h + (_BLK - 1), -_BLK)
    cum_hp = plsc.cumsum(hp)
    offv = cum_hp - hp

    @pl.when(wid == 0)
    def _():
        off_v[...] = cum_hp
        pltpu.sync_copy(off_v, off_out)

    nf_v[...] = offv + pre
    for v in range(_CHUNK // 16):
        ev = eid_all[pl.ds(wid * _CHUNK + v * 16, 16)]
        base = plsc.load_gather(nf_v, [ev])
        rank = jnp.zeros((16,), jnp.int32)
        for e in range(_E):
            mask = ev == e
            incl = plsc.cumsum(jnp.where(mask, 1, 0))
            rank = jnp.where(mask, incl - 1, rank)
        pos_v[v // 4, pl.ds((v % 4) * 16, 16)] = base + rank
        plsc.addupdate_scatter(nf_v, [ev], ones16)
    pltpu.sync_copy(pos_v, pos_out.at[wid])

    for k in range(2):
        tok_base = (wid % (_NW // 2)) * _CHUNK + k * _HALF
        pltpu.sync_copy(x_hbm.at[pl.ds(tok_base, _HALF)], xbuf)
        pltpu.async_copy(xbuf, xs_out.at[pos_v.at[k]], sem).wait()


def _ffn_body(ex_ref, x_ref, gw_ref, uw_ref, dw_ref, y_ref):
    t = pl.program_id(0)
    e = ex_ref[t]

    @pl.when(e < _E)
    def _():
        x = x_ref[...]
        g = jax.lax.dot_general(
            x, gw_ref[0], (((1,), (1,)), ((), ())),
            preferred_element_type=jnp.float32)
        u = jax.lax.dot_general(
            x, uw_ref[0], (((1,), (1,)), ((), ())),
            preferred_element_type=jnp.float32)
        h = (g * jax.lax.logistic(g)) * u
        y_ref[...] = jax.lax.dot_general(
            h, dw_ref[0], (((1,), (1,)), ((), ())),
            preferred_element_type=jnp.float32)


_SUB = 16
_NSUB = (_N // _NW) // _SUB


def _combine_kernel(ys_hbm, pos_hbm, p12_hbm, out_hbm,
                    pos_v, p_v, ya, yb, ob, sem):
    wid = lax.axis_index("s") * 2 + lax.axis_index("c")
    iota = lax.iota(jnp.int32, 16)
    tok0 = wid * (_N // _NW)
    pltpu.sync_copy(pos_hbm.at[wid // 2, wid % 2], pos_v.at[0])
    pltpu.sync_copy(pos_hbm.at[_NW // 2 + wid // 2, wid % 2], pos_v.at[1])
    pltpu.sync_copy(p12_hbm.at[0, pl.ds(tok0, _N // _NW)], p_v.at[0])
    pltpu.sync_copy(p12_hbm.at[1, pl.ds(tok0, _N // _NW)], p_v.at[1])
    for s in range(_NSUB):
        cpa = pltpu.async_copy(
            ys_hbm.at[pos_v.at[0, pl.ds(s * _SUB, _SUB)]], ya, sem)
        cpb = pltpu.async_copy(
            ys_hbm.at[pos_v.at[1, pl.ds(s * _SUB, _SUB)]], yb, sem)
        cpa.wait()
        cpb.wait()
        pav = p_v[0, pl.ds(s * _SUB, 16)]
        pbv = p_v[1, pl.ds(s * _SUB, 16)]
        pa_l = [jnp.sum(jnp.where(iota == j, pav, 0.0)) for j in range(_SUB)]
        pb_l = [jnp.sum(jnp.where(iota == j, pbv, 0.0)) for j in range(_SUB)]

        def cbody(c, carry):
            sl = pl.ds(c * 16, 16)
            for j in range(_SUB):
                ob[j, sl] = pa_l[j] * ya[j, sl] + pb_l[j] * yb[j, sl]
            return carry

        lax.fori_loop(0, _D // 16, cbody, 0)
        pltpu.sync_copy(ob, out_hbm.at[pl.ds(tok0 + s * _SUB, _SUB)])


def _block_experts(cum_hp):
    jj = jnp.arange(_NBLKP, dtype=jnp.int32)[:, None] * _BLK
    ex = jnp.sum((cum_hp[None, :_E] <= jj).astype(jnp.int32), axis=1)
    return ex.astype(jnp.int32)


_sc_mesh = plsc.VectorSubcoreMesh(core_axis_name="c", subcore_axis_name="s")

_dispatch = functools.partial(
    pl.kernel, _dispatch_kernel, mesh=_sc_mesh,
    compiler_params=pltpu.CompilerParams(needs_layout_passes=False),
    out_type=(
        jax.ShapeDtypeStruct((_NW, 2, _HALF), jnp.int32),
        jax.ShapeDtypeStruct((16,), jnp.int32),
        jax.ShapeDtypeStruct((_AP, _D), jnp.float32),
    ),
    scratch_types=[
        pltpu.VMEM((_A,), jnp.int32),
        pltpu.VMEM((16,), jnp.int32),
        pltpu.VMEM((2, _HALF), jnp.int32),
        pltpu.VMEM((16,), jnp.int32),
        pltpu.VMEM((_HALF, _D), jnp.float32),
        pltpu.SemaphoreType.DMA,
    ],
)

_combine = functools.partial(
    pl.kernel, _combine_kernel, mesh=_sc_mesh,
    compiler_params=pltpu.CompilerParams(needs_layout_passes=False),
    out_type=jax.ShapeDtypeStruct((_N, _D), jnp.float32),
    scratch_types=[
        pltpu.VMEM((2, _N // _NW), jnp.int32),
        pltpu.VMEM((2, _N // _NW), jnp.float32),
        pltpu.VMEM((_SUB, _D), jnp.float32),
        pltpu.VMEM((_SUB, _D), jnp.float32),
        pltpu.VMEM((_SUB, _D), jnp.float32),
        pltpu.SemaphoreType.DMA,
    ],
)


@jax.jit
def kernel(hidden_states, router_w, gate_w, up_w, down_w):
    x = hidden_states.reshape(_N, _D)
    logits, i12, p12 = pl.pallas_call(
        _router_body,
        out_shape=(
            jax.ShapeDtypeStruct((_N, _E), jnp.float32),
            jax.ShapeDtypeStruct((_TOPK, _N), jnp.int32),
            jax.ShapeDtypeStruct((_TOPK, _N), jnp.float32),
        ),
    )(x, router_w)

    eid = i12.reshape(_A)
    pos3, cum_hp, x_sorted = _dispatch()(eid, x)
    ex = _block_experts(cum_hp)

    y_sorted = pl.pallas_call(
        _ffn_body,
        grid_spec=pltpu.PrefetchScalarGridSpec(
            num_scalar_prefetch=1,
            grid=(_NBLKP,),
            in_specs=[
                pl.BlockSpec((_BLK, _D), lambda t, e: (t, 0)),
                pl.BlockSpec((1, _F, _D),
                             lambda t, e: (jnp.minimum(e[t], _E - 1), 0, 0)),
                pl.BlockSpec((1, _F, _D),
                             lambda t, e: (jnp.minimum(e[t], _E - 1), 0, 0)),
                pl.BlockSpec((1, _D, _F),
                             lambda t, e: (jnp.minimum(e[t], _E - 1), 0, 0)),
            ],
            out_specs=pl.BlockSpec((_BLK, _D), lambda t, e: (t, 0)),
        ),
        out_shape=jax.ShapeDtypeStruct((_AP, _D), jnp.float32),
    )(ex, x_sorted, gate_w, up_w, down_w)

    out = _combine()(y_sorted, pos3, p12)

    return out.reshape(_B, _S, _D), logits


# speedup vs baseline: 1.6957x; 1.0750x over previous
import functools

import jax
import jax.numpy as jnp
from jax import lax
from jax.experimental import pallas as pl
from jax.experimental.pallas import tpu as pltpu
from jax.experimental.pallas import tpu_sc as plsc

_B, _S, _D = 1, 2048, 1024
_E, _TOPK, _F = 8, 2, 512
_N = _B * _S
_A = _N * _TOPK

_BLK = 512
_AP = _A + _E * _BLK
_NBLKP = _AP // _BLK

_NW = 32
_CHUNK = _A // _NW
_HALF = _CHUNK // 2

_NEG = float("-inf")


def _router_body(x_ref, rw_ref, logits_ref, i12_ref, p12_ref):
    x = x_ref[...]
    rw = rw_ref[...]
    logits = jax.lax.dot_general(
        x, rw, (((1,), (1,)), ((), ())), preferred_element_type=jnp.float32)
    logits_ref[...] = logits
    logits_t = jax.lax.dot_general(
        rw, x, (((1,), (1,)), ((), ())), preferred_element_type=jnp.float32)
    idx = jax.lax.broadcasted_iota(jnp.int32, (_E, _N), 0)
    m1 = jnp.max(logits_t, axis=0, keepdims=True)
    i1 = jnp.min(jnp.where(logits_t == m1, idx, _E), axis=0, keepdims=True)
    l2 = jnp.where(idx == i1, _NEG, logits_t)
    m2 = jnp.max(l2, axis=0, keepdims=True)
    i2 = jnp.min(jnp.where(l2 == m2, idx, _E), axis=0, keepdims=True)
    t = jnp.exp(m2 - m1)
    p1 = 1.0 / (1.0 + t)
    i12_ref[...] = jnp.concatenate([i1, i2], axis=0)
    p12_ref[...] = jnp.concatenate([p1, t * p1], axis=0)


def _histo_chunk(eid_all, hist_ref, ones16, c):
    for v in range(_CHUNK // 16):
        ev = eid_all[pl.ds(c * _CHUNK + v * 16, 16)]
        plsc.addupdate_scatter(hist_ref, [ev], ones16)


def _dispatch_kernel(eid_hbm, x_hbm, pos_out, off_out, xs_out,
                     eid_v, nf_v, pos_v, off_v, xbuf, sem):
    wid = lax.axis_index("s") * 2 + lax.axis_index("c")
    ones16 = jnp.ones((16,), jnp.int32)

    pltpu.sync_copy(eid_hbm, eid_v)
    eid_all = eid_v

    nf_v[...] = jnp.zeros((16,), jnp.int32)

    def hbody(c, carry):
        _histo_chunk(eid_all, nf_v, ones16, c)
        return carry

    lax.fori_loop(0, wid, hbody, 0)
    pre = nf_v[...]
    lax.fori_loop(wid, _NW, hbody, 0)
    h = nf_v[...]
    hp = jnp.bitwise_and(h + (_BLK - 1), -_BLK)
    cum_hp = plsc.cumsum(hp)
    offv = cum_hp - hp

    @pl.when(wid == 0)
    def _():
        off_v[...] = cum_hp
        pltpu.sync_copy(off_v, off_out)

    nf_v[...] = offv + pre
    for v in range(_CHUNK // 16):
        ev = eid_all[pl.ds(wid * _CHUNK + v * 16, 16)]
        base = plsc.load_gather(nf_v, [ev])
        rank = jnp.zeros((16,), jnp.int32)
        for e in range(_E):
            mask = ev == e
            incl = plsc.cumsum(jnp.where(mask, 1, 0))
            rank = jnp.where(mask, incl - 1, rank)
        pos_v[v // 4, pl.ds((v % 4) * 16, 16)] = base + rank
        plsc.addupdate_scatter(nf_v, [ev], ones16)
    pltpu.sync_copy(pos_v, pos_out.at[wid])

    for k in range(2):
        tok_base = (wid % (_NW // 2)) * _CHUNK + k * _HALF
        pltpu.sync_copy(x_hbm.at[pl.ds(tok_base, _HALF)], xbuf)
        pltpu.async_copy(xbuf, xs_out.at[pos_v.at[k]], sem).wait()


def _ffn_body(ex_ref, x_ref, gw_ref, uw_ref, dw_ref, y_ref):
    t = pl.program_id(0)
    e = ex_ref[t]

    @pl.when(e < _E)
    def _():
        x = x_ref[...]
        g = jax.lax.dot_general(
            x, gw_ref[0], (((1,), (1,)), ((), ())),
            preferred_element_type=jnp.float32)
        u = jax.lax.dot_general(
            x, uw_ref[0], (((1,), (1,)), ((), ())),
            preferred_element_type=jnp.float32)
        h = (g * jax.lax.logistic(g)) * u
        y_ref[...] = jax.lax.dot_general(
            h, dw_ref[0], (((1,), (1,)), ((), ())),
            preferred_element_type=jnp.float32)


_SUB = 16
_NSUB = (_N // _NW) // _SUB


def _combine_kernel(ys_hbm, pos_hbm, p12_hbm, out_hbm,
                    pos_v, p_v, ya0, yb0, ya1, yb1, ob0, ob1,
                    sem_g, sem_s):
    wid = lax.axis_index("s") * 2 + lax.axis_index("c")
    iota = lax.iota(jnp.int32, 16)
    tok0 = wid * (_N // _NW)
    pltpu.sync_copy(pos_hbm.at[wid // 2, wid % 2], pos_v.at[0])
    pltpu.sync_copy(pos_hbm.at[_NW // 2 + wid // 2, wid % 2], pos_v.at[1])
    pltpu.sync_copy(p12_hbm.at[0, pl.ds(tok0, _N // _NW)], p_v.at[0])
    pltpu.sync_copy(p12_hbm.at[1, pl.ds(tok0, _N // _NW)], p_v.at[1])
    ya = [ya0, ya1]
    yb = [yb0, yb1]
    ob = [ob0, ob1]

    def gathers(s):
        b = s % 2
        return (
            pltpu.async_copy(
                ys_hbm.at[pos_v.at[0, pl.ds(s * _SUB, _SUB)]], ya[b], sem_g),
            pltpu.async_copy(
                ys_hbm.at[pos_v.at[1, pl.ds(s * _SUB, _SUB)]], yb[b], sem_g),
        )

    def store(s):
        return pltpu.async_copy(
            ob[s % 2], out_hbm.at[pl.ds(tok0 + s * _SUB, _SUB)], sem_s)

    cps = {0: gathers(0)}
    sts = {}
    for s in range(_NSUB):
        b = s % 2
        ca, cb = cps[s]
        ca.wait()
        cb.wait()
        if s + 1 < _NSUB:
            cps[s + 1] = gathers(s + 1)
        if s - 2 >= 0:
            sts[s - 2].wait()
        pav = p_v[0, pl.ds(s * _SUB, 16)]
        pbv = p_v[1, pl.ds(s * _SUB, 16)]
        pa_l = [jnp.sum(jnp.where(iota == j, pav, 0.0)) for j in range(_SUB)]
        pb_l = [jnp.sum(jnp.where(iota == j, pbv, 0.0)) for j in range(_SUB)]

        def cbody(c, carry, b=b, pa_l=pa_l, pb_l=pb_l):
            sl = pl.ds(c * 16, 16)
            for j in range(_SUB):
                ob[b][j, sl] = pa_l[j] * ya[b][j, sl] + pb_l[j] * yb[b][j, sl]
            return carry

        lax.fori_loop(0, _D // 16, cbody, 0)
        sts[s] = store(s)
    for s in range(max(0, _NSUB - 2), _NSUB):
        sts[s].wait()


def _block_experts(cum_hp):
    jj = jnp.arange(_NBLKP, dtype=jnp.int32)[:, None] * _BLK
    ex = jnp.sum((cum_hp[None, :_E] <= jj).astype(jnp.int32), axis=1)
    return ex.astype(jnp.int32)


_sc_mesh = plsc.VectorSubcoreMesh(core_axis_name="c", subcore_axis_name="s")

_dispatch = functools.partial(
    pl.kernel, _dispatch_kernel, mesh=_sc_mesh,
    compiler_params=pltpu.CompilerParams(needs_layout_passes=False),
    out_type=(
        jax.ShapeDtypeStruct((_NW, 2, _HALF), jnp.int32),
        jax.ShapeDtypeStruct((16,), jnp.int32),
        jax.ShapeDtypeStruct((_AP, _D), jnp.float32),
    ),
    scratch_types=[
        pltpu.VMEM((_A,), jnp.int32),
        pltpu.VMEM((16,), jnp.int32),
        pltpu.VMEM((2, _HALF), jnp.int32),
        pltpu.VMEM((16,), jnp.int32),
        pltpu.VMEM((_HALF, _D), jnp.float32),
        pltpu.SemaphoreType.DMA,
    ],
)

_combine = functools.partial(
    pl.kernel, _combine_kernel, mesh=_sc_mesh,
    compiler_params=pltpu.CompilerParams(needs_layout_passes=False),
    out_type=jax.ShapeDtypeStruct((_N, _D), jnp.float32),
    scratch_types=[
        pltpu.VMEM((2, _N // _NW), jnp.int32),
        pltpu.VMEM((2, _N // _NW), jnp.float32),
        pltpu.VMEM((_SUB, _D), jnp.float32),
        pltpu.VMEM((_SUB, _D), jnp.float32),
        pltpu.VMEM((_SUB, _D), jnp.float32),
        pltpu.VMEM((_SUB, _D), jnp.float32),
        pltpu.VMEM((_SUB, _D), jnp.float32),
        pltpu.VMEM((_SUB, _D), jnp.float32),
        pltpu.SemaphoreType.DMA,
        pltpu.SemaphoreType.DMA,
    ],
)


@jax.jit
def kernel(hidden_states, router_w, gate_w, up_w, down_w):
    x = hidden_states.reshape(_N, _D)
    logits, i12, p12 = pl.pallas_call(
        _router_body,
        out_shape=(
            jax.ShapeDtypeStruct((_N, _E), jnp.float32),
            jax.ShapeDtypeStruct((_TOPK, _N), jnp.int32),
            jax.ShapeDtypeStruct((_TOPK, _N), jnp.float32),
        ),
    )(x, router_w)

    eid = i12.reshape(_A)
    pos3, cum_hp, x_sorted = _dispatch()(eid, x)
    ex = _block_experts(cum_hp)

    y_sorted = pl.pallas_call(
        _ffn_body,
        grid_spec=pltpu.PrefetchScalarGridSpec(
            num_scalar_prefetch=1,
            grid=(_NBLKP,),
            in_specs=[
                pl.BlockSpec((_BLK, _D), lambda t, e: (t, 0)),
                pl.BlockSpec((1, _F, _D),
                             lambda t, e: (jnp.minimum(e[t], _E - 1), 0, 0)),
                pl.BlockSpec((1, _F, _D),
                             lambda t, e: (jnp.minimum(e[t], _E - 1), 0, 0)),
                pl.BlockSpec((1, _D, _F),
                             lambda t, e: (jnp.minimum(e[t], _E - 1), 0, 0)),
            ],
            out_specs=pl.BlockSpec((_BLK, _D), lambda t, e: (t, 0)),
        ),
        out_shape=jax.ShapeDtypeStruct((_AP, _D), jnp.float32),
    )(ex, x_sorted, gate_w, up_w, down_w)

    out = _combine()(y_sorted, pos3, p12)

    return out.reshape(_B, _S, _D), logits


# speedup vs baseline: 1.7179x; 1.0131x over previous
import functools

import jax
import jax.numpy as jnp
from jax import lax
from jax.experimental import pallas as pl
from jax.experimental.pallas import tpu as pltpu
from jax.experimental.pallas import tpu_sc as plsc

_B, _S, _D = 1, 2048, 1024
_E, _TOPK, _F = 8, 2, 512
_N = _B * _S
_A = _N * _TOPK

_BLK = 512
_AP = _A + _E * _BLK
_NBLKP = _AP // _BLK

_NW = 32
_CHUNK = _A // _NW
_HALF = _CHUNK // 2

_NEG = float("-inf")


def _router_body(x_ref, rw_ref, logits_ref, i12_ref, p12_ref):
    x = x_ref[...]
    rw = rw_ref[...]
    logits = jax.lax.dot_general(
        x, rw, (((1,), (1,)), ((), ())), preferred_element_type=jnp.float32)
    logits_ref[...] = logits
    logits_t = jax.lax.dot_general(
        rw, x, (((1,), (1,)), ((), ())), preferred_element_type=jnp.float32)
    idx = jax.lax.broadcasted_iota(jnp.int32, (_E, _N), 0)
    m1 = jnp.max(logits_t, axis=0, keepdims=True)
    i1 = jnp.min(jnp.where(logits_t == m1, idx, _E), axis=0, keepdims=True)
    l2 = jnp.where(idx == i1, _NEG, logits_t)
    m2 = jnp.max(l2, axis=0, keepdims=True)
    i2 = jnp.min(jnp.where(l2 == m2, idx, _E), axis=0, keepdims=True)
    t = jnp.exp(m2 - m1)
    p1 = 1.0 / (1.0 + t)
    i12_ref[...] = jnp.concatenate([i1, i2], axis=0)
    p12_ref[...] = jnp.concatenate([p1, t * p1], axis=0)


def _histo_chunk(eid_all, hist_ref, ones16, c):
    for v in range(_CHUNK // 16):
        ev = eid_all[pl.ds(c * _CHUNK + v * 16, 16)]
        plsc.addupdate_scatter(hist_ref, [ev], ones16)


def _dispatch_kernel(eid_hbm, x_hbm, pos_out, off_out, xs_out,
                     eid_v, nf_v, pos_v, off_v, xb0, xb1, xb2, sem_r, sem_c):
    wid = lax.axis_index("s") * 2 + lax.axis_index("c")
    ones16 = jnp.ones((16,), jnp.int32)

    pltpu.sync_copy(eid_hbm, eid_v)
    eid_all = eid_v

    xb = [xb0, xb1, xb2]

    def read_q(q):
        tok_base = (wid % (_NW // 2)) * _CHUNK + q * 32
        return pltpu.async_copy(
            x_hbm.at[pl.ds(tok_base, 32)], xb[q % 3], sem_r)

    rds = {q: read_q(q) for q in range(3)}

    nf_v[...] = jnp.zeros((16,), jnp.int32)

    def hbody(c, carry):
        _histo_chunk(eid_all, nf_v, ones16, c)
        return carry

    lax.fori_loop(0, wid, hbody, 0)
    pre = nf_v[...]
    lax.fori_loop(wid, _NW, hbody, 0)
    h = nf_v[...]
    hp = jnp.bitwise_and(h + (_BLK - 1), -_BLK)
    cum_hp = plsc.cumsum(hp)
    offv = cum_hp - hp

    @pl.when(wid == 0)
    def _():
        off_v[...] = cum_hp
        pltpu.sync_copy(off_v, off_out)

    nf_v[...] = offv + pre
    for v in range(_CHUNK // 16):
        ev = eid_all[pl.ds(wid * _CHUNK + v * 16, 16)]
        base = plsc.load_gather(nf_v, [ev])
        rank = jnp.zeros((16,), jnp.int32)
        for e in range(_E):
            mask = ev == e
            incl = plsc.cumsum(jnp.where(mask, 1, 0))
            rank = jnp.where(mask, incl - 1, rank)
        pos_v[v // 2, pl.ds((v % 2) * 16, 16)] = base + rank
        plsc.addupdate_scatter(nf_v, [ev], ones16)
    pltpu.sync_copy(pos_v, pos_out.at[wid])

    scs = {}
    for q in range(4):
        rds[q].wait()
        scs[q] = pltpu.async_copy(xb[q % 3], xs_out.at[pos_v.at[q]], sem_c)
        if q + 3 < 4:
            scs[q].wait()
            rds[q + 3] = read_q(q + 3)
    for q in range(1, 4):
        scs[q].wait()


def _ffn_body(ex_ref, x_ref, gw_ref, uw_ref, dw_ref, y_ref):
    t = pl.program_id(0)
    e = ex_ref[t]

    @pl.when(e < _E)
    def _():
        x = x_ref[...]
        g = jax.lax.dot_general(
            x, gw_ref[0], (((1,), (1,)), ((), ())),
            preferred_element_type=jnp.float32)
        u = jax.lax.dot_general(
            x, uw_ref[0], (((1,), (1,)), ((), ())),
            preferred_element_type=jnp.float32)
        h = (g * jax.lax.logistic(g)) * u
        y_ref[...] = jax.lax.dot_general(
            h, dw_ref[0], (((1,), (1,)), ((), ())),
            preferred_element_type=jnp.float32)


_SUB = 16
_NSUB = (_N // _NW) // _SUB


def _combine_kernel(ys_hbm, pos_hbm, p12_hbm, out_hbm,
                    pos_v, p_v, ya0, yb0, ya1, yb1, ob0, ob1,
                    sem_g, sem_s):
    wid = lax.axis_index("s") * 2 + lax.axis_index("c")
    iota = lax.iota(jnp.int32, 16)
    tok0 = wid * (_N // _NW)
    for k in range(2):
        for hh in range(2):
            pltpu.sync_copy(
                pos_hbm.at[k * (_NW // 2) + wid // 2, (wid % 2) * 2 + hh],
                pos_v.at[k, pl.ds(hh * 32, 32)])
    pltpu.sync_copy(p12_hbm.at[0, pl.ds(tok0, _N // _NW)], p_v.at[0])
    pltpu.sync_copy(p12_hbm.at[1, pl.ds(tok0, _N // _NW)], p_v.at[1])
    ya = [ya0, ya1]
    yb = [yb0, yb1]
    ob = [ob0, ob1]

    def gathers(s):
        b = s % 2
        return (
            pltpu.async_copy(
                ys_hbm.at[pos_v.at[0, pl.ds(s * _SUB, _SUB)]], ya[b], sem_g),
            pltpu.async_copy(
                ys_hbm.at[pos_v.at[1, pl.ds(s * _SUB, _SUB)]], yb[b], sem_g),
        )

    def store(s):
        return pltpu.async_copy(
            ob[s % 2], out_hbm.at[pl.ds(tok0 + s * _SUB, _SUB)], sem_s)

    cps = {0: gathers(0)}
    sts = {}
    for s in range(_NSUB):
        b = s % 2
        ca, cb = cps[s]
        ca.wait()
        cb.wait()
        if s + 1 < _NSUB:
            cps[s + 1] = gathers(s + 1)
        if s - 2 >= 0:
            sts[s - 2].wait()
        pav = p_v[0, pl.ds(s * _SUB, 16)]
        pbv = p_v[1, pl.ds(s * _SUB, 16)]
        pa_l = [jnp.sum(jnp.where(iota == j, pav, 0.0)) for j in range(_SUB)]
        pb_l = [jnp.sum(jnp.where(iota == j, pbv, 0.0)) for j in range(_SUB)]

        def cbody(c, carry, b=b, pa_l=pa_l, pb_l=pb_l):
            sl = pl.ds(c * 16, 16)
            for j in range(_SUB):
                ob[b][j, sl] = pa_l[j] * ya[b][j, sl] + pb_l[j] * yb[b][j, sl]
            return carry

        lax.fori_loop(0, _D // 16, cbody, 0)
        sts[s] = store(s)
    for s in range(max(0, _NSUB - 2), _NSUB):
        sts[s].wait()


def _block_experts(cum_hp):
    jj = jnp.arange(_NBLKP, dtype=jnp.int32)[:, None] * _BLK
    ex = jnp.sum((cum_hp[None, :_E] <= jj).astype(jnp.int32), axis=1)
    return ex.astype(jnp.int32)


_sc_mesh = plsc.VectorSubcoreMesh(core_axis_name="c", subcore_axis_name="s")

_dispatch = functools.partial(
    pl.kernel, _dispatch_kernel, mesh=_sc_mesh,
    compiler_params=pltpu.CompilerParams(needs_layout_passes=False),
    out_type=(
        jax.ShapeDtypeStruct((_NW, 4, 32), jnp.int32),
        jax.ShapeDtypeStruct((16,), jnp.int32),
        jax.ShapeDtypeStruct((_AP, _D), jnp.float32),
    ),
    scratch_types=[
        pltpu.VMEM((_A,), jnp.int32),
        pltpu.VMEM((16,), jnp.int32),
        pltpu.VMEM((4, 32), jnp.int32),
        pltpu.VMEM((16,), jnp.int32),
        pltpu.VMEM((32, _D), jnp.float32),
        pltpu.VMEM((32, _D), jnp.float32),
        pltpu.VMEM((32, _D), jnp.float32),
        pltpu.SemaphoreType.DMA,
        pltpu.SemaphoreType.DMA,
    ],
)

_combine = functools.partial(
    pl.kernel, _combine_kernel, mesh=_sc_mesh,
    compiler_params=pltpu.CompilerParams(needs_layout_passes=False),
    out_type=jax.ShapeDtypeStruct((_N, _D), jnp.float32),
    scratch_types=[
        pltpu.VMEM((2, _N // _NW), jnp.int32),
        pltpu.VMEM((2, _N // _NW), jnp.float32),
        pltpu.VMEM((_SUB, _D), jnp.float32),
        pltpu.VMEM((_SUB, _D), jnp.float32),
        pltpu.VMEM((_SUB, _D), jnp.float32),
        pltpu.VMEM((_SUB, _D), jnp.float32),
        pltpu.VMEM((_SUB, _D), jnp.float32),
        pltpu.VMEM((_SUB, _D), jnp.float32),
        pltpu.SemaphoreType.DMA,
        pltpu.SemaphoreType.DMA,
    ],
)


@jax.jit
def kernel(hidden_states, router_w, gate_w, up_w, down_w):
    x = hidden_states.reshape(_N, _D)
    logits, i12, p12 = pl.pallas_call(
        _router_body,
        out_shape=(
            jax.ShapeDtypeStruct((_N, _E), jnp.float32),
            jax.ShapeDtypeStruct((_TOPK, _N), jnp.int32),
            jax.ShapeDtypeStruct((_TOPK, _N), jnp.float32),
        ),
    )(x, router_w)

    eid = i12.reshape(_A)
    pos3, cum_hp, x_sorted = _dispatch()(eid, x)
    ex = _block_experts(cum_hp)

    y_sorted = pl.pallas_call(
        _ffn_body,
        grid_spec=pltpu.PrefetchScalarGridSpec(
            num_scalar_prefetch=1,
            grid=(_NBLKP,),
            in_specs=[
                pl.BlockSpec((_BLK, _D), lambda t, e: (t, 0)),
                pl.BlockSpec((1, _F, _D),
                             lambda t, e: (jnp.minimum(e[t], _E - 1), 0, 0)),
                pl.BlockSpec((1, _F, _D),
                             lambda t, e: (jnp.minimum(e[t], _E - 1), 0, 0)),
                pl.BlockSpec((1, _D, _F),
                             lambda t, e: (jnp.minimum(e[t], _E - 1), 0, 0)),
            ],
            out_specs=pl.BlockSpec((_BLK, _D), lambda t, e: (t, 0)),
        ),
        out_shape=jax.ShapeDtypeStruct((_AP, _D), jnp.float32),
    )(ex, x_sorted, gate_w, up_w, down_w)

    out = _combine()(y_sorted, pos3, p12)

    return out.reshape(_B, _S, _D), logits
